# Initial kernel scaffold; baseline (speedup 1.0000x reference)
import jax
import jax.numpy as jnp
from jax.experimental import pallas as pl


def kernel(inputs, token_table, position_table):
    raise NotImplementedError("write your pallas kernel here")


# speedup vs baseline: 1.1805x; 1.1805x over previous
import functools

import jax
import jax.numpy as jnp
from jax import lax
from jax.experimental import pallas as pl
from jax.experimental.pallas import tpu as pltpu
from jax.experimental.pallas import tpu_sc as plsc

DIM = 32
LANES = 16
NC, NS = 2, 16
NW = NC * NS


def _sc_body(n_seq_per_w, seq, idx_hbm, tok_hbm, pos_hbm, out_hbm,
             pos_v, idx_v, rows_v, sem):
    wid = lax.axis_index("s") * NC + lax.axis_index("c")
    pltpu.sync_copy(pos_hbm, pos_v)

    def one_seq(r, carry):
        base = (wid * n_seq_per_w + r) * seq
        pltpu.sync_copy(idx_hbm.at[pl.ds(base, seq)], idx_v)
        pltpu.async_copy(tok_hbm.at[idx_v], rows_v, sem).wait()

        def add_row(i, c):
            for j in range(DIM // LANES):
                sl = pl.ds(j * LANES, LANES)
                rows_v[i, sl] = rows_v[i, sl] + pos_v[i, sl]
            return c

        lax.fori_loop(0, seq, add_row, 0)
        pltpu.sync_copy(rows_v, out_hbm.at[pl.ds(base, seq)])
        return carry

    lax.fori_loop(0, n_seq_per_w, one_seq, 0)


def kernel(inputs, token_table, position_table):
    b, l = inputs.shape
    n = b * l
    flat_idx = inputs.reshape(n).astype(jnp.int32)
    n_seq_per_w = b * l // (NW * l)

    mesh = plsc.VectorSubcoreMesh(core_axis_name="c", subcore_axis_name="s",
                                  num_cores=NC, num_subcores=NS)
    out = pl.kernel(
        functools.partial(_sc_body, n_seq_per_w, l),
        out_type=jax.ShapeDtypeStruct((n, DIM), jnp.float32),
        mesh=mesh,
        scratch_types=[
            pltpu.VMEM((l, DIM), jnp.float32),
            pltpu.VMEM((l,), jnp.int32),
            pltpu.VMEM((l, DIM), jnp.float32),
            pltpu.SemaphoreType.DMA,
        ],
        compiler_params=pltpu.CompilerParams(use_tc_tiling_on_sc=False),
    )(flat_idx, token_table, position_table)
    return out.reshape(b, l, DIM)


# speedup vs baseline: 1.3971x; 1.1834x over previous
import functools

import jax
import jax.numpy as jnp
from jax import lax
from jax.experimental import pallas as pl
from jax.experimental.pallas import tpu as pltpu
from jax.experimental.pallas import tpu_sc as plsc

DIM = 32
LANES = 16
NC, NS = 2, 16
NW = NC * NS
NBUF = 2
UNROLL = 8


def _sc_body(n_seq_per_w, seq, idx_hbm, tok_hbm, pos_hbm, out_hbm,
             pos_v, idx_v, rows_v, g0, g1, o0, o1):
    gsems = [g0, g1]
    osems = [o0, o1]
    wid = lax.axis_index("s") * NC + lax.axis_index("c")
    w0 = wid * n_seq_per_w
    pltpu.sync_copy(pos_hbm, pos_v)

    for b in range(NBUF):
        pltpu.sync_copy(idx_hbm.at[pl.ds((w0 + b) * seq, seq)], idx_v.at[b])
        pltpu.async_copy(tok_hbm.at[idx_v.at[b]], rows_v.at[b], gsems[b])

    n_outer = n_seq_per_w // NBUF

    def outer(r, carry):
        for b in range(NBUF):
            s = r * NBUF + b
            base = (w0 + s) * seq
            pltpu.make_async_copy(
                tok_hbm.at[idx_v.at[b]], rows_v.at[b], gsems[b]).wait()

            def add_rows(i, c, b=b):
                for dj in range(UNROLL):
                    row = i * UNROLL + dj
                    for j in range(DIM // LANES):
                        sl = pl.ds(j * LANES, LANES)
                        rows_v[b, row, sl] = rows_v[b, row, sl] + pos_v[row, sl]
                return c

            lax.fori_loop(0, seq // UNROLL, add_rows, 0)
            pltpu.async_copy(rows_v.at[b], out_hbm.at[pl.ds(base, seq)],
                             osems[b])

            @pl.when(r < n_outer - 1)
            def _(b=b, s=s, base=base):
                nbase = (w0 + s + NBUF) * seq
                pltpu.sync_copy(idx_hbm.at[pl.ds(nbase, seq)], idx_v.at[b])
                pltpu.make_async_copy(
                    rows_v.at[b], out_hbm.at[pl.ds(base, seq)],
                    osems[b]).wait()
                pltpu.async_copy(tok_hbm.at[idx_v.at[b]], rows_v.at[b],
                                 gsems[b])
        return carry

    lax.fori_loop(0, n_outer, outer, 0)
    for b in range(NBUF):
        pltpu.make_async_copy(rows_v.at[b], out_hbm.at[pl.ds(0, seq)],
                              osems[b]).wait()


def kernel(inputs, token_table, position_table):
    b, l = inputs.shape
    n = b * l
    flat_idx = inputs.reshape(n).astype(jnp.int32)
    n_seq_per_w = b // NW

    mesh = plsc.VectorSubcoreMesh(core_axis_name="c", subcore_axis_name="s",
                                  num_cores=NC, num_subcores=NS)
    out = pl.kernel(
        functools.partial(_sc_body, n_seq_per_w, l),
        out_type=jax.ShapeDtypeStruct((n, DIM), jnp.float32),
        mesh=mesh,
        scratch_types=[
            pltpu.VMEM((l, DIM), jnp.float32),
            pltpu.VMEM((NBUF, l), jnp.int32),
            pltpu.VMEM((NBUF, l, DIM), jnp.float32),
            pltpu.SemaphoreType.DMA,
            pltpu.SemaphoreType.DMA,
            pltpu.SemaphoreType.DMA,
            pltpu.SemaphoreType.DMA,
        ],
        compiler_params=pltpu.CompilerParams(use_tc_tiling_on_sc=False),
    )(flat_idx, token_table, position_table)
    return out.reshape(b, l, DIM)


# speedup vs baseline: 1.4910x; 1.0673x over previous
import functools

import jax
import jax.numpy as jnp
from jax import lax
from jax.experimental import pallas as pl
from jax.experimental.pallas import tpu as pltpu
from jax.experimental.pallas import tpu_sc as plsc

DIM = 32
LANES = 16
NC, NS = 2, 16
NW = NC * NS
NBUF = 2
CHUNK = 4
UNROLL = 8


def _sc_body(n_seq_per_w, seq, idx_hbm, tok_hbm, pos_hbm, out_hbm,
             pos_v, idx_v, rows_v, g0, g1, o0, o1):
    gsems = [g0, g1]
    osems = [o0, o1]
    wid = lax.axis_index("s") * NC + lax.axis_index("c")
    w0 = wid * n_seq_per_w
    crows = CHUNK * seq
    pltpu.sync_copy(pos_hbm, pos_v)

    for b in range(NBUF):
        pltpu.sync_copy(idx_hbm.at[pl.ds(w0 * seq + b * crows, crows)],
                        idx_v.at[b])
        pltpu.async_copy(tok_hbm.at[idx_v.at[b]], rows_v.at[b], gsems[b])

    n_outer = n_seq_per_w // CHUNK // NBUF

    def outer(r, carry):
        for b in range(NBUF):
            s = r * NBUF + b
            base = w0 * seq + s * crows
            pltpu.make_async_copy(
                tok_hbm.at[idx_v.at[b]], rows_v.at[b], gsems[b]).wait()

            def add_rows(i, c, b=b):
                for dj in range(UNROLL):
                    row = i * UNROLL + dj
                    for j in range(DIM // LANES):
                        sl = pl.ds(j * LANES, LANES)
                        p = pos_v[row, sl]
                        for q in range(CHUNK):
                            rq = q * seq + row
                            rows_v[b, rq, sl] = rows_v[b, rq, sl] + p
                return c

            lax.fori_loop(0, seq // UNROLL, add_rows, 0)
            pltpu.async_copy(rows_v.at[b], out_hbm.at[pl.ds(base, crows)],
                             osems[b])

            @pl.when(r < n_outer - 1)
            def _(b=b, s=s, base=base):
                nbase = base + NBUF * crows
                pltpu.sync_copy(idx_hbm.at[pl.ds(nbase, crows)], idx_v.at[b])
                pltpu.make_async_copy(
                    rows_v.at[b], out_hbm.at[pl.ds(base, crows)],
                    osems[b]).wait()
                pltpu.async_copy(tok_hbm.at[idx_v.at[b]], rows_v.at[b],
                                 gsems[b])
        return carry

    lax.fori_loop(0, n_outer, outer, 0)
    for b in range(NBUF):
        pltpu.make_async_copy(rows_v.at[b], out_hbm.at[pl.ds(0, crows)],
                              osems[b]).wait()


def kernel(inputs, token_table, position_table):
    b, l = inputs.shape
    n = b * l
    flat_idx = inputs.reshape(n).astype(jnp.int32)
    n_seq_per_w = b // NW

    mesh = plsc.VectorSubcoreMesh(core_axis_name="c", subcore_axis_name="s",
                                  num_cores=NC, num_subcores=NS)
    out = pl.kernel(
        functools.partial(_sc_body, n_seq_per_w, l),
        out_type=jax.ShapeDtypeStruct((n, DIM), jnp.float32),
        mesh=mesh,
        scratch_types=[
            pltpu.VMEM((l, DIM), jnp.float32),
            pltpu.VMEM((NBUF, CHUNK * l), jnp.int32),
            pltpu.VMEM((NBUF, CHUNK * l, DIM), jnp.float32),
            pltpu.SemaphoreType.DMA,
            pltpu.SemaphoreType.DMA,
            pltpu.SemaphoreType.DMA,
            pltpu.SemaphoreType.DMA,
        ],
        compiler_params=pltpu.CompilerParams(use_tc_tiling_on_sc=False),
    )(flat_idx, token_table, position_table)
    return out.reshape(b, l, DIM)
